# baseline (speedup 1.0000x reference)
import functools

import jax
import jax.numpy as jnp
from jax import lax
from jax.experimental import pallas as pl
from jax.experimental.pallas import tpu as pltpu
from jax.experimental.pallas import tpu_sc as plsc

N = 10000
E = 320000
D = 128
NC = 2
NS = 16
NW = NC * NS
NP = 10112
RPW = NP // NS
EPW = E // NW
KCH = 78
PH = 3
KPP = KCH // PH
TAIL = EPW - KCH * 128

_mesh = plsc.VectorSubcoreMesh(
    core_axis_name="c", subcore_axis_name="s", num_cores=NC, num_subcores=NS
)


def _wid():
    return lax.axis_index("s") * NC + lax.axis_index("c")


@functools.partial(
    pl.kernel,
    out_type=jax.ShapeDtypeStruct((NC, NP, 16), jnp.float32),
    mesh=_mesh,
    scratch_types=[
        pltpu.VMEM((PH, KPP, 128), jnp.int32),
        pltpu.VMEM((TAIL,), jnp.int32),
        pltpu.VMEM((128, 16), jnp.float32),
        pltpu.VMEM_SHARED((NP, 16), jnp.float32),
        pltpu.SemaphoreType.DMA,
    ],
)
def _sc_deg(dst_hbm, dstt_hbm, ones_hbm, out_hbm, idx_d, tidx_d, ones_v, acc,
            sem):
    c = lax.axis_index("c")
    s = lax.axis_index("s")
    wid = _wid()
    pltpu.sync_copy(dst_hbm.at[wid], idx_d)
    pltpu.sync_copy(dstt_hbm.at[wid], tidx_d)
    pltpu.sync_copy(ones_hbm, ones_v)
    for i in range(RPW // 128):
        pltpu.sync_copy(ones_hbm, acc.at[pl.ds(s * RPW + i * 128, 128)])
    pltpu.sync_copy(
        ones_hbm.at[pl.ds(0, RPW % 128)],
        acc.at[pl.ds(s * RPW + (RPW // 128) * 128, RPW % 128)],
    )
    plsc.subcore_barrier()

    descs = []
    for ph in range(PH):
        for k in range(KPP):
            descs.append(pltpu.async_copy(
                ones_v, acc.at[idx_d.at[ph, k]], sem, add=True))
    for d in descs:
        d.wait()
    pltpu.sync_copy(ones_v.at[pl.ds(0, TAIL)], acc.at[tidx_d], add=True)
    plsc.subcore_barrier()
    pltpu.sync_copy(acc.at[pl.ds(s * RPW, RPW)], out_hbm.at[c, pl.ds(s * RPW, RPW)])


@functools.partial(
    pl.kernel,
    out_type=jax.ShapeDtypeStruct((NC, NP, D), jnp.float32),
    mesh=_mesh,
    scratch_types=[
        pltpu.VMEM((KPP, 128), jnp.int32),
        pltpu.VMEM((KPP, 128), jnp.int32),
        pltpu.VMEM((TAIL,), jnp.int32),
        pltpu.VMEM((TAIL,), jnp.int32),
        pltpu.VMEM((128, D), jnp.float32),
        pltpu.VMEM((128, D), jnp.float32),
        pltpu.VMEM((TAIL, D), jnp.float32),
        pltpu.VMEM_SHARED((NP, D), jnp.float32),
        pltpu.SemaphoreType.DMA,
        pltpu.SemaphoreType.DMA,
        pltpu.SemaphoreType.DMA,
        pltpu.SemaphoreType.DMA,
    ],
)
def _sc_prop(u_hbm, src_hbm, dst_hbm, srct_hbm, dstt_hbm, out_hbm,
             idx_s, idx_d, tidx_s, tidx_d, rows0, rows1, trows, acc,
             semg0, semg1, sems0, sems1):
    c = lax.axis_index("c")
    s = lax.axis_index("s")
    wid = _wid()
    pltpu.sync_copy(srct_hbm.at[wid], tidx_s)
    pltpu.sync_copy(dstt_hbm.at[wid], tidx_d)
    pltpu.sync_copy(u_hbm.at[pl.ds(s * RPW, RPW)], acc.at[pl.ds(s * RPW, RPW)])
    plsc.subcore_barrier()

    bufs = (rows0, rows1)
    gsems = (semg0, semg1)
    ssems = (sems0, sems1)
    for ph in range(PH):
        pltpu.sync_copy(src_hbm.at[wid, ph], idx_s)
        pltpu.sync_copy(dst_hbm.at[wid, ph], idx_d)
        gd = [None, None]
        sd = [None, None]
        for k in range(KPP):
            p = k & 1
            if sd[p] is not None:
                sd[p].wait()
            gd[p] = pltpu.async_copy(u_hbm.at[idx_s.at[k]], bufs[p], gsems[p])
            if k > 0:
                q = (k - 1) & 1
                gd[q].wait()
                sd[q] = pltpu.async_copy(
                    bufs[q], acc.at[idx_d.at[k - 1]], ssems[q], add=True)
        p = (KPP - 1) & 1
        gd[p].wait()
        sd[p] = pltpu.async_copy(
            bufs[p], acc.at[idx_d.at[KPP - 1]], ssems[p], add=True)
        sd[0].wait()
        sd[1].wait()

    pltpu.async_copy(u_hbm.at[tidx_s], trows, semg0).wait()
    pltpu.sync_copy(trows, acc.at[tidx_d], add=True)
    plsc.subcore_barrier()
    pltpu.sync_copy(acc.at[pl.ds(s * RPW, RPW)], out_hbm.at[c, pl.ds(s * RPW, RPW)])


def _tc_pre_body(dega, degb, x, dinv_ref, u0_ref):
    deg = dega[:, 0:1] + degb[:, 0:1] - 1.0
    dinv = lax.rsqrt(deg)
    dinv_ref[...] = dinv
    u0_ref[...] = dinv * x[...]


_TGRID = 8
_TBLK = NP // _TGRID


def _tc_pre(dega, degb, x):
    return pl.pallas_call(
        _tc_pre_body,
        grid=(_TGRID,),
        in_specs=[
            pl.BlockSpec((_TBLK, 16), lambda i: (i, 0)),
            pl.BlockSpec((_TBLK, 16), lambda i: (i, 0)),
            pl.BlockSpec((_TBLK, D), lambda i: (i, 0)),
        ],
        out_specs=[
            pl.BlockSpec((_TBLK, 1), lambda i: (i, 0)),
            pl.BlockSpec((_TBLK, D), lambda i: (i, 0)),
        ],
        out_shape=[
            jax.ShapeDtypeStruct((NP, 1), jnp.float32),
            jax.ShapeDtypeStruct((NP, D), jnp.float32),
        ],
    )(dega, degb, x)


def _tc_mid_body(s0a, s0b, u0, dinv, w1, w2, b1, u1_ref):
    y = dinv[...] * (s0a[...] + s0b[...] - u0[...])
    w12 = jnp.dot(w1[...], w2[...], precision=lax.Precision.HIGHEST)
    c = jnp.dot(b1[...], w2[...], precision=lax.Precision.HIGHEST)
    z = jnp.dot(y, w12, precision=lax.Precision.HIGHEST) + c
    u1_ref[...] = dinv[...] * z


def _tc_mid(s0a, s0b, u0, dinv, w1, w2, b1):
    return pl.pallas_call(
        _tc_mid_body,
        grid=(_TGRID,),
        in_specs=[
            pl.BlockSpec((_TBLK, D), lambda i: (i, 0)),
            pl.BlockSpec((_TBLK, D), lambda i: (i, 0)),
            pl.BlockSpec((_TBLK, D), lambda i: (i, 0)),
            pl.BlockSpec((_TBLK, 1), lambda i: (i, 0)),
            pl.BlockSpec((128, 2 * D), lambda i: (0, 0)),
            pl.BlockSpec((2 * D, D), lambda i: (0, 0)),
            pl.BlockSpec((1, 2 * D), lambda i: (0, 0)),
        ],
        out_specs=pl.BlockSpec((_TBLK, D), lambda i: (i, 0)),
        out_shape=jax.ShapeDtypeStruct((NP, D), jnp.float32),
    )(s0a, s0b, u0, dinv, w1, w2, b1)


def _tc_post_body(s1a, s1b, u1, dinv, b2, gamma, beta, out_ref):
    h2 = dinv[...] * (s1a[...] + s1b[...] - u1[...]) + b2[...]
    row = lax.broadcasted_iota(jnp.int32, (NP, 1), 0)
    valid = (row < N).astype(jnp.float32)
    h2v = h2 * valid
    mean = jnp.sum(h2v, axis=0, keepdims=True) * (1.0 / N)
    cent = (h2 - mean) * valid
    var = jnp.sum(cent * cent, axis=0, keepdims=True) * (1.0 / N)
    out_ref[...] = (h2 - mean) * lax.rsqrt(var + 1e-5) * gamma[...] + beta[...]


def _tc_post(s1a, s1b, u1, dinv, b2, gamma, beta):
    return pl.pallas_call(
        _tc_post_body,
        out_shape=jax.ShapeDtypeStruct((NP, D), jnp.float32),
    )(s1a, s1b, u1, dinv, b2, gamma, beta)


def kernel(edge_index, emb_weight, W1, b1, W2, b2, bn_gamma, bn_beta):
    src = edge_index[0].astype(jnp.int32).reshape(NW, EPW)
    dst = edge_index[1].astype(jnp.int32).reshape(NW, EPW)
    src_main = src[:, : KCH * 128].reshape(NW, PH, KPP, 128)
    dst_main = dst[:, : KCH * 128].reshape(NW, PH, KPP, 128)
    src_tail = src[:, KCH * 128:]
    dst_tail = dst[:, KCH * 128:]

    ones128 = jnp.ones((128, 16), jnp.float32)
    xpad = jnp.zeros((NP, D), jnp.float32).at[:N].set(emb_weight)

    deg_parts = _sc_deg(dst_main, dst_tail, ones128)
    dinv, u0 = _tc_pre(deg_parts[0], deg_parts[1], xpad)
    s0 = _sc_prop(u0, src_main, dst_main, src_tail, dst_tail)
    u1 = _tc_mid(s0[0], s0[1], u0, dinv, W1, W2, b1.reshape(1, -1))
    s1 = _sc_prop(u1, src_main, dst_main, src_tail, dst_tail)
    out = _tc_post(
        s1[0], s1[1], u1, dinv,
        b2.reshape(1, -1), bn_gamma.reshape(1, -1), bn_beta.reshape(1, -1),
    )
    return out[:N]
